# baseline (speedup 1.0000x reference)
import functools

import jax
import jax.numpy as jnp
from jax import lax
from jax.experimental import pallas as pl
from jax.experimental.pallas import tpu as pltpu
from jax.experimental.pallas import tpu_sc as plsc

NINP = 64
CHUNK = 128


def _make_emb(total, cpw, nw):
    mesh = plsc.VectorSubcoreMesh(core_axis_name="c", subcore_axis_name="s")

    @functools.partial(
        pl.kernel,
        mesh=mesh,
        out_type=jax.ShapeDtypeStruct((total, NINP), jnp.float32),
        scratch_types=[
            pltpu.VMEM((CHUNK,), jnp.int32),
            pltpu.VMEM((CHUNK, NINP), jnp.float32),
            pltpu.SemaphoreType.DMA,
        ],
        compiler_params=pltpu.CompilerParams(use_tc_tiling_on_sc=False),
    )
    def emb(idx_hbm, table_hbm, out_hbm, idx_v, rows_v, sem):
        wid = lax.axis_index("s") * 2 + lax.axis_index("c")

        def body(c, carry):
            chunk = wid * cpw + c
            pltpu.sync_copy(idx_hbm.at[chunk], idx_v)
            pltpu.async_copy(table_hbm.at[idx_v], rows_v, sem).wait()
            pltpu.sync_copy(rows_v, out_hbm.at[pl.ds(chunk * CHUNK, CHUNK)])
            return carry

        lax.fori_loop(0, cpw, body, 0)

    return emb


def kernel(input, weight):
    batch, hist = input.shape
    total = batch * hist
    nw = 32
    cpw = total // (CHUNK * nw)
    idx = input.reshape(nw * cpw, CHUNK).astype(jnp.int32)
    out = _make_emb(total, cpw, nw)(idx, weight)
    return out.reshape(batch, hist, NINP)


# speedup vs baseline: 1.1949x; 1.1949x over previous
import functools

import jax
import jax.numpy as jnp
from jax import lax
from jax.experimental import pallas as pl
from jax.experimental.pallas import tpu as pltpu
from jax.experimental.pallas import tpu_sc as plsc

NINP = 64
CHUNK = 128
GK = 4
GROUP = GK * CHUNK
NBUF = 3


def _make_emb(total, cpw, nw):
    ngroups = cpw // GK
    mesh = plsc.VectorSubcoreMesh(core_axis_name="c", subcore_axis_name="s")

    @functools.partial(
        pl.kernel,
        mesh=mesh,
        out_type=jax.ShapeDtypeStruct((total, NINP), jnp.float32),
        scratch_types=[
            pltpu.VMEM((cpw, CHUNK), jnp.int32),
            pltpu.VMEM((NBUF, GROUP, NINP), jnp.float32),
            pltpu.SemaphoreType.DMA,
        ]
        + [pltpu.SemaphoreType.DMA] * NBUF
        + [pltpu.SemaphoreType.DMA] * NBUF,
        compiler_params=pltpu.CompilerParams(use_tc_tiling_on_sc=False),
    )
    def emb(idx_hbm, table_hbm, out_hbm, idx_v, rows_v, sem_i, *sems):
        sem_g = sems[:NBUF]
        sem_w = sems[NBUF:]
        wid = lax.axis_index("s") * 2 + lax.axis_index("c")
        base_chunk = wid * cpw

        def fire(g, b):
            for j in range(GK):
                pltpu.async_copy(
                    table_hbm.at[idx_v.at[g * GK + j]],
                    rows_v.at[b, pl.ds(j * CHUNK, CHUNK)],
                    sem_g[b],
                )

        def drain(b):
            for j in range(GK):
                pltpu.make_async_copy(
                    table_hbm.at[idx_v.at[0]],
                    rows_v.at[b, pl.ds(j * CHUNK, CHUNK)],
                    sem_g[b],
                ).wait()

        def issue_write(g, b):
            pltpu.async_copy(
                rows_v.at[b],
                out_hbm.at[pl.ds((base_chunk + g * GK) * CHUNK, GROUP)],
                sem_w[b],
            )

        def wait_write(b):
            pltpu.make_async_copy(
                rows_v.at[b],
                out_hbm.at[pl.ds(0, GROUP)],
                sem_w[b],
            ).wait()

        pltpu.async_copy(idx_hbm.at[pl.ds(base_chunk, cpw)], idx_v, sem_i).wait()

        fire(0, 0)

        def body(g, carry):
            for bn in range(NBUF):

                @pl.when(((g + 1) % NBUF == bn) & (g + 1 < ngroups))
                def _():
                    @pl.when(g + 1 >= NBUF)
                    def _():
                        wait_write(bn)

                    fire(g + 1, bn)

            for b in range(NBUF):

                @pl.when(g % NBUF == b)
                def _():
                    drain(b)
                    issue_write(g, b)

            return carry

        lax.fori_loop(0, ngroups, body, 0)
        for b in range(NBUF):
            wait_write(b)

    return emb


def kernel(input, weight):
    batch, hist = input.shape
    total = batch * hist
    nw = 32
    cpw = total // (CHUNK * nw)
    idx = input.reshape(nw * cpw, CHUNK).astype(jnp.int32)
    out = _make_emb(total, cpw, nw)(idx, weight)
    return out.reshape(batch, hist, NINP)
